# Initial kernel scaffold; baseline (speedup 1.0000x reference)
import jax
import jax.numpy as jnp
from jax.experimental import pallas as pl


def kernel(x_num, x_cat, col_mean, col_std, W_num, b_num, emb, W_out, b_out):
    raise NotImplementedError("write your pallas kernel here")


# speedup vs baseline: 9.9196x; 9.9196x over previous
import functools

import jax
import jax.numpy as jnp
import numpy as np
from jax import lax
from jax.experimental import pallas as pl
from jax.experimental.pallas import tpu as pltpu
from jax.experimental.pallas import tpu_sc as plsc

B = 16384
NUM_COLS = 13
CAT_COLS = 26
VOCAB = 100000
C = 32
L = 16
NC, NS = 2, 16
NW = NC * NS
BPW = B // NW
RPC = 4
IPC = RPC * CAT_COLS
NCHUNK = BPW // RPC
IDX_PER_W = BPW * CAT_COLS
PAT = 208


def _sc_body(emb_hbm, xcat_hbm, xnum_hbm, pat_hbm, par_hbm, out_hbm,
             idx_v, xnum_v, pat_v, par_v, buf, tmat_v, out_v, sem0, sem1):
    wid = lax.axis_index("s") * NC + lax.axis_index("c")
    base = wid * BPW

    pltpu.sync_copy(xcat_hbm.at[pl.ds(wid * IDX_PER_W, IDX_PER_W)], idx_v)
    pltpu.sync_copy(xnum_hbm.at[pl.ds(base * L, BPW * L)], xnum_v)
    pltpu.sync_copy(pat_hbm, pat_v)
    pltpu.sync_copy(par_hbm, par_v)

    pats = [pat_v[pl.ds(j * L, L)] for j in range(PAT // L)]

    @pl.loop(0, IDX_PER_W // PAT)
    def _(g):
        gb = g * PAT
        for j in range(PAT // L):
            s = gb + j * L
            idx_v[pl.ds(s, L)] = idx_v[pl.ds(s, L)] + pats[j]

    vs_lo = par_v[0, 0:L]
    vs_hi = par_v[1, 0:L]
    scl = par_v[2, 0:L]
    rowbase = lax.iota(jnp.int32, L) * L

    def chunk_copy(c, slot, sem):
        return pltpu.make_async_copy(
            emb_hbm.at[idx_v.at[pl.ds(c * IPC, IPC)]], buf.at[slot], sem)

    chunk_copy(0, 0, sem0).start()

    @pl.loop(0, NCHUNK, step=4)
    def _(c0):
        for s in range(4):
            c = c0 + s
            slot = s % 2
            sem = sem0 if slot == 0 else sem1
            nsem = sem1 if slot == 0 else sem0

            @pl.when(c + 1 < NCHUNK)
            def _():
                chunk_copy(c + 1, 1 - slot, nsem).start()

            chunk_copy(c, slot, sem).wait()

            for r in range(RPC):
                rb = r * CAT_COLS
                acc0 = buf[slot, rb, 0:L]
                acc1 = buf[slot, rb, L:C]
                for k in range(1, CAT_COLS):
                    acc0 = acc0 + buf[slot, rb + k, 0:L]
                    acc1 = acc1 + buf[slot, rb + k, L:C]
                row = c * RPC + r
                t = (acc0 * vs_lo + acc1 * vs_hi
                     + xnum_v[pl.ds(row * L, L)] * scl)
                tmat_v[pl.ds((s * RPC + r) * L, L)] = t

        ovec = plsc.load_gather(tmat_v, [rowbase])
        for col in range(1, L):
            ovec = ovec + plsc.load_gather(tmat_v, [rowbase + col])
        out_v[pl.ds(c0 * RPC, L)] = ovec

    pltpu.sync_copy(out_v, out_hbm.at[pl.ds(base, BPW)])


@jax.jit
def _run(emb_flat, xcat_flat, xnum_pad, pat, par):
    mesh = plsc.VectorSubcoreMesh(core_axis_name="c", subcore_axis_name="s")
    f = functools.partial(
        pl.kernel,
        out_type=jax.ShapeDtypeStruct((B,), jnp.float32),
        mesh=mesh,
        compiler_params=pltpu.CompilerParams(
            needs_layout_passes=False, use_tc_tiling_on_sc=False),
        scratch_types=[
            pltpu.VMEM((IDX_PER_W,), jnp.int32),
            pltpu.VMEM((BPW * L,), jnp.float32),
            pltpu.VMEM((PAT,), jnp.int32),
            pltpu.VMEM((3, L), jnp.float32),
            pltpu.VMEM((2, IPC, C), jnp.float32),
            pltpu.VMEM((L * L,), jnp.float32),
            pltpu.VMEM((BPW,), jnp.float32),
            pltpu.SemaphoreType.DMA,
            pltpu.SemaphoreType.DMA,
        ],
    )(_sc_body)
    return f(emb_flat, xcat_flat, xnum_pad, pat, par)


def kernel(x_num, x_cat, col_mean, col_std, W_num, b_num, emb, W_out, b_out):
    v = W_out[:, 0]
    u = W_num @ v
    scl = u / col_std
    ncols = NUM_COLS + CAT_COLS
    const = (jnp.sum(b_num @ v) - jnp.sum(col_mean * scl)) / ncols + b_out[0]

    scl_full = jnp.concatenate([
        scl / ncols,
        jnp.reshape(const, (1,)),
        jnp.zeros((L - NUM_COLS - 1,), jnp.float32),
    ])
    par = jnp.stack([v[0:L] / ncols, v[L:C] / ncols, scl_full])
    pat = jnp.asarray((np.arange(PAT) % CAT_COLS) * VOCAB, dtype=jnp.int32)

    emb_flat = emb.reshape(CAT_COLS * VOCAB, C)
    xcat_flat = x_cat.reshape(-1)
    xnum_pad = jnp.concatenate([
        x_num,
        jnp.ones((B, 1), jnp.float32),
        jnp.zeros((B, L - NUM_COLS - 1), jnp.float32),
    ], axis=1).reshape(-1)
    return _run(emb_flat, xcat_flat, xnum_pad, pat, par)


# speedup vs baseline: 10.1488x; 1.0231x over previous
import functools

import jax
import jax.numpy as jnp
import numpy as np
from jax import lax
from jax.experimental import pallas as pl
from jax.experimental.pallas import tpu as pltpu
from jax.experimental.pallas import tpu_sc as plsc

B = 16384
NUM_COLS = 13
CAT_COLS = 26
VOCAB = 100000
C = 32
L = 16
NC, NS = 2, 16
NW = NC * NS
BPW = B // NW
RPC = 4
IPC = RPC * CAT_COLS
NCHUNK = BPW // RPC
IDX_PER_W = BPW * CAT_COLS
PAT = 208


NBUF = 4


def _sc_body(emb_hbm, xcat_hbm, xnum_hbm, pat_hbm, par_hbm, out_hbm,
             idx_v, xnum_v, pat_v, par_v, buf, tmat_v, out_v, *sems):
    wid = lax.axis_index("s") * NC + lax.axis_index("c")
    base = wid * BPW

    pltpu.sync_copy(xcat_hbm.at[pl.ds(wid * IDX_PER_W, IDX_PER_W)], idx_v)
    pltpu.sync_copy(xnum_hbm.at[pl.ds(base * L, BPW * L)], xnum_v)
    pltpu.sync_copy(pat_hbm, pat_v)
    pltpu.sync_copy(par_hbm, par_v)

    pats = [pat_v[pl.ds(j * L, L)] for j in range(PAT // L)]

    @pl.loop(0, IDX_PER_W // PAT)
    def _(g):
        gb = g * PAT
        for j in range(PAT // L):
            s = gb + j * L
            idx_v[pl.ds(s, L)] = idx_v[pl.ds(s, L)] + pats[j]

    vs_lo = par_v[0, 0:L]
    vs_hi = par_v[1, 0:L]
    scl = par_v[2, 0:L]
    rowbase = lax.iota(jnp.int32, L) * L

    def chunk_copy(c, slot):
        return pltpu.make_async_copy(
            emb_hbm.at[idx_v.at[pl.ds(c * IPC, IPC)]], buf.at[slot],
            sems[slot])

    for c in range(NBUF - 1):
        chunk_copy(c, c).start()

    @pl.loop(0, NCHUNK, step=4)
    def _(c0):
        for s in range(4):
            c = c0 + s
            slot = s % NBUF

            @pl.when(c + NBUF - 1 < NCHUNK)
            def _():
                chunk_copy(c + NBUF - 1, (s + NBUF - 1) % NBUF).start()

            chunk_copy(c, slot).wait()

            for r in range(RPC):
                rb = r * CAT_COLS
                acc0 = buf[slot, rb, 0:L]
                acc1 = buf[slot, rb, L:C]
                for k in range(1, CAT_COLS):
                    acc0 = acc0 + buf[slot, rb + k, 0:L]
                    acc1 = acc1 + buf[slot, rb + k, L:C]
                row = c * RPC + r
                t = (acc0 * vs_lo + acc1 * vs_hi
                     + xnum_v[pl.ds(row * L, L)] * scl)
                tmat_v[pl.ds((s * RPC + r) * L, L)] = t

        ovec = plsc.load_gather(tmat_v, [rowbase])
        for col in range(1, L):
            ovec = ovec + plsc.load_gather(tmat_v, [rowbase + col])
        out_v[pl.ds(c0 * RPC, L)] = ovec

    pltpu.sync_copy(out_v, out_hbm.at[pl.ds(base, BPW)])


@jax.jit
def _run(emb_flat, xcat_flat, xnum_pad, pat, par):
    mesh = plsc.VectorSubcoreMesh(core_axis_name="c", subcore_axis_name="s")
    f = functools.partial(
        pl.kernel,
        out_type=jax.ShapeDtypeStruct((B,), jnp.float32),
        mesh=mesh,
        compiler_params=pltpu.CompilerParams(
            needs_layout_passes=False, use_tc_tiling_on_sc=False),
        scratch_types=[
            pltpu.VMEM((IDX_PER_W,), jnp.int32),
            pltpu.VMEM((BPW * L,), jnp.float32),
            pltpu.VMEM((PAT,), jnp.int32),
            pltpu.VMEM((3, L), jnp.float32),
            pltpu.VMEM((NBUF, IPC, C), jnp.float32),
            pltpu.VMEM((L * L,), jnp.float32),
            pltpu.VMEM((BPW,), jnp.float32),
        ] + [pltpu.SemaphoreType.DMA] * NBUF,
    )(_sc_body)
    return f(emb_flat, xcat_flat, xnum_pad, pat, par)


def kernel(x_num, x_cat, col_mean, col_std, W_num, b_num, emb, W_out, b_out):
    v = W_out[:, 0]
    u = W_num @ v
    scl = u / col_std
    ncols = NUM_COLS + CAT_COLS
    const = (jnp.sum(b_num @ v) - jnp.sum(col_mean * scl)) / ncols + b_out[0]

    scl_full = jnp.concatenate([
        scl / ncols,
        jnp.reshape(const, (1,)),
        jnp.zeros((L - NUM_COLS - 1,), jnp.float32),
    ])
    par = jnp.stack([v[0:L] / ncols, v[L:C] / ncols, scl_full])
    pat = jnp.asarray((np.arange(PAT) % CAT_COLS) * VOCAB, dtype=jnp.int32)

    emb_flat = emb.reshape(CAT_COLS * VOCAB, C)
    xcat_flat = x_cat.reshape(-1)
    xnum_pad = jnp.concatenate([
        x_num,
        jnp.ones((B, 1), jnp.float32),
        jnp.zeros((B, L - NUM_COLS - 1), jnp.float32),
    ], axis=1).reshape(-1)
    return _run(emb_flat, xcat_flat, xnum_pad, pat, par)
